# baseline (speedup 1.0000x reference)
import functools

import jax
import jax.numpy as jnp
from jax import lax
from jax.experimental import pallas as pl
from jax.experimental.pallas import tpu as pltpu
from jax.experimental.pallas import tpu_sc as plsc

NUM_CORES = 2
NUM_SUBCORES = 16
NUM_WORKERS = NUM_CORES * NUM_SUBCORES

MAX_IDX_PER_STREAM = 128
BLOCK = 4


def _sc_embed(idx, table):
    batch, hist = idx.shape
    vocab, dim = table.shape
    rows_per_worker = batch // NUM_WORKERS
    n_blocks = rows_per_worker // BLOCK
    assert n_blocks % 2 == 0
    segs = []
    off = 0
    while off < hist:
        n = min(MAX_IDX_PER_STREAM, hist - off)
        segs.append((off, n))
        off += n

    mesh = plsc.VectorSubcoreMesh(core_axis_name="c", subcore_axis_name="s")

    @functools.partial(
        pl.kernel,
        out_type=jax.ShapeDtypeStruct((batch, hist, dim), jnp.float32),
        mesh=mesh,
        scratch_types=[
            pltpu.VMEM_SHARED((vocab, dim), jnp.float32),
            pltpu.VMEM((rows_per_worker, hist), jnp.int32),
            pltpu.VMEM((BLOCK, hist, dim), jnp.float32),
            pltpu.VMEM((BLOCK, hist, dim), jnp.float32),
            pltpu.SemaphoreType.DMA,
            pltpu.SemaphoreType.DMA,
            pltpu.SemaphoreType.DMA,
            pltpu.SemaphoreType.DMA,
        ],
        compiler_params=pltpu.CompilerParams(use_tc_tiling_on_sc=False),
    )
    def k(table_hbm, idx_hbm, out_hbm, table_sh, idx_v, rows0, rows1,
          g0sem, g1sem, o0sem, o1sem):
        wid = lax.axis_index("s") * NUM_CORES + lax.axis_index("c")
        row_base = wid * rows_per_worker

        @pl.when(lax.axis_index("s") == 0)
        def _():
            pltpu.sync_copy(table_hbm, table_sh)

        plsc.subcore_barrier()
        pltpu.sync_copy(idx_hbm.at[pl.ds(row_base, rows_per_worker)], idx_v)

        def fire_gather(blk, rows, sem):
            for q in range(BLOCK):
                for off, n in segs:
                    pltpu.async_copy(
                        table_sh.at[idx_v.at[blk * BLOCK + q, pl.ds(off, n)]],
                        rows.at[q, pl.ds(off, n)],
                        sem,
                    )

        def wait_gather(rows, sem):
            pltpu.make_async_copy(
                out_hbm.at[pl.ds(0, BLOCK)], rows, sem).wait()

        def fire_out(blk, rows, sem):
            pltpu.async_copy(
                rows, out_hbm.at[pl.ds(row_base + blk * BLOCK, BLOCK)], sem)

        def wait_out(rows, sem):
            pltpu.make_async_copy(
                rows, out_hbm.at[pl.ds(0, BLOCK)], sem).wait()

        fire_gather(0, rows0, g0sem)
        fire_gather(1, rows1, g1sem)

        def body(i, carry):
            b0 = 2 * i
            b1 = b0 + 1
            wait_gather(rows0, g0sem)
            fire_out(b0, rows0, o0sem)
            wait_gather(rows1, g1sem)
            fire_out(b1, rows1, o1sem)
            wait_out(rows0, o0sem)
            fire_gather(b0 + 2, rows0, g0sem)
            wait_out(rows1, o1sem)
            fire_gather(b1 + 2, rows1, g1sem)
            return carry

        lax.fori_loop(0, n_blocks // 2 - 1, body, 0)

        b0 = n_blocks - 2
        wait_gather(rows0, g0sem)
        fire_out(b0, rows0, o0sem)
        wait_gather(rows1, g1sem)
        fire_out(b0 + 1, rows1, o1sem)
        wait_out(rows0, o0sem)
        wait_out(rows1, o1sem)

    return k(table, idx)


def kernel(inputs, table):
    return _sc_embed(inputs.astype(jnp.int32), table)
